# baseline (speedup 1.0000x reference)
import functools

import jax
import jax.numpy as jnp
from jax import lax
from jax.experimental import pallas as pl
from jax.experimental.pallas import tpu as pltpu
from jax.experimental.pallas import tpu_sc as plsc

N = 10000
NP = 10240
E = 320000
D = 128
CH = 128
NW = 32
NCH = 80
EP = NW * NCH * CH
RPT = NP // 16
TCB = 2048
GRID = NP // TCB


def _sc_mesh():
    return plsc.VectorSubcoreMesh(core_axis_name="c", subcore_axis_name="s")


def _deg_body(dst_hbm, ones_hbm, zeros_hbm, out_hbm, dst_v, ones_v, acc,
              s0, s1, s2, s3):
    sems = (s0, s1, s2, s3)
    cid = lax.axis_index("c")
    sid = lax.axis_index("s")
    wid = sid * 2 + cid
    base = sid * RPT
    pltpu.sync_copy(zeros_hbm.at[pl.ds(base, RPT)], acc.at[pl.ds(base, RPT)])
    pltpu.sync_copy(dst_hbm.at[wid], dst_v)
    pltpu.sync_copy(ones_hbm, ones_v)
    plsc.subcore_barrier()

    def s_fire(j, b):
        pltpu.async_copy(ones_v, acc.at[dst_v.at[j]], sems[b], add=True)

    def s_wait(j, b):
        pltpu.make_async_copy(ones_v, acc.at[dst_v.at[j]], sems[b]).wait()

    for c in range(4):
        s_fire(c, c)

    def body(t, carry):
        for b in range(4):
            j = 4 + 4 * t + b
            s_wait(j - 4, b)
            s_fire(j, b)
        return carry

    lax.fori_loop(0, (NCH - 4) // 4, body, 0)
    for j in range(NCH - 4, NCH):
        s_wait(j, j % 4)

    plsc.subcore_barrier()
    pltpu.sync_copy(acc.at[pl.ds(base, RPT)], out_hbm.at[cid, pl.ds(base, RPT)])


def _make_deg_kernel():
    return pl.kernel(
        _deg_body,
        out_type=jax.ShapeDtypeStruct((2, NP, 16), jnp.float32),
        mesh=_sc_mesh(),
        scratch_types=[
            pltpu.VMEM((NCH, CH), jnp.int32),
            pltpu.VMEM((CH, 16), jnp.float32),
            pltpu.VMEM_SHARED((NP, 16), jnp.float32),
            pltpu.SemaphoreType.DMA,
            pltpu.SemaphoreType.DMA,
            pltpu.SemaphoreType.DMA,
            pltpu.SemaphoreType.DMA,
        ],
        compiler_params=pltpu.CompilerParams(use_tc_tiling_on_sc=False),
    )


DEPTH = 4
NBUF = 2 * DEPTH


def _layer_body(y_hbm, src_hbm, dst_hbm, zeros_hbm, out_hbm,
                src_v, dst_v, *rest):
    bufs = rest[:NBUF]
    acc = rest[NBUF]
    gsem = rest[NBUF + 1:2 * NBUF + 1]
    ssem = rest[2 * NBUF + 1:]
    cid = lax.axis_index("c")
    sid = lax.axis_index("s")
    wid = sid * 2 + cid
    base = sid * RPT
    pltpu.sync_copy(zeros_hbm.at[pl.ds(base, RPT)], acc.at[pl.ds(base, RPT)])
    pltpu.sync_copy(src_hbm.at[wid], src_v)
    pltpu.sync_copy(dst_hbm.at[wid], dst_v)
    plsc.subcore_barrier()

    y_mine = y_hbm.at[cid]

    def g_fire(j, bi):
        pltpu.async_copy(y_mine.at[src_v.at[j]], bufs[bi], gsem[bi])

    def g_wait(j, bi):
        pltpu.make_async_copy(y_mine.at[src_v.at[j]], bufs[bi], gsem[bi]).wait()

    def s_fire(j, bi):
        pltpu.async_copy(bufs[bi], acc.at[dst_v.at[j]], ssem[bi], add=True)

    def s_wait(j, bi):
        pltpu.make_async_copy(bufs[bi], acc.at[dst_v.at[j]], ssem[bi]).wait()

    for c in range(DEPTH):
        g_fire(c, c)
    for j in range(DEPTH):
        g_wait(j, j)
        s_fire(j, j)
        g_fire(j + DEPTH, (j + DEPTH) % NBUF)

    def body(t, carry):
        j0 = DEPTH + t * NBUF
        for b in range(NBUF):
            j = j0 + b
            g_wait(j, (DEPTH + b) % NBUF)
            s_fire(j, (DEPTH + b) % NBUF)
            s_wait(j - DEPTH, b)
            g_fire(j + DEPTH, b)
        return carry

    lax.fori_loop(0, (NCH - 2 * DEPTH) // NBUF, body, 0)

    for j in range(NCH - DEPTH, NCH):
        bi = j % NBUF
        g_wait(j, bi)
        s_fire(j, bi)
        s_wait(j - DEPTH, (j - DEPTH) % NBUF)
    for j in range(NCH - DEPTH, NCH):
        s_wait(j, j % NBUF)

    plsc.subcore_barrier()
    pltpu.sync_copy(acc.at[pl.ds(base, RPT)], out_hbm.at[cid, pl.ds(base, RPT)])


def _make_layer_kernel(hp):
    return pl.kernel(
        _layer_body,
        out_type=jax.ShapeDtypeStruct((2, NP, hp), jnp.float32),
        mesh=_sc_mesh(),
        scratch_types=(
            [pltpu.VMEM((NCH, CH), jnp.int32),
             pltpu.VMEM((NCH, CH), jnp.int32)]
            + [pltpu.VMEM((CH, hp), jnp.float32) for _ in range(NBUF)]
            + [pltpu.VMEM_SHARED((NP, hp), jnp.float32)]
            + [pltpu.SemaphoreType.DMA for _ in range(2 * NBUF)]
        ),
        compiler_params=pltpu.CompilerParams(use_tc_tiling_on_sc=False),
    )


def _dis_of(pd_blk):
    deg = 1.0 + pd_blk[0, :, 0:1] + pd_blk[1, :, 0:1]
    return lax.rsqrt(deg)


def _tc_first_body(x_ref, w_ref, pd_ref, y_ref):
    dis = _dis_of(pd_ref[...])
    y_ref[...] = jnp.dot(x_ref[...], w_ref[...],
                         preferred_element_type=jnp.float32) * dis


def _tc_mid_body(y_ref, p_ref, pd_ref, b_ref, w_ref, o_ref):
    dis = _dis_of(pd_ref[...])
    p = p_ref[...]
    s = y_ref[...] + p[0] + p[1]
    h = jnp.maximum(s * dis + b_ref[0:1, :], 0.0)
    o_ref[...] = jnp.dot(h, w_ref[...],
                         preferred_element_type=jnp.float32) * dis


def _tc_last_body(y_ref, p_ref, pd_ref, b_ref, o_ref):
    dis = _dis_of(pd_ref[...])
    p = p_ref[...]
    z = (y_ref[...] + p[0] + p[1]) * dis + b_ref[0:1, :]
    mask = lax.broadcasted_iota(jnp.int32, z.shape, 1) < 10
    zm = jnp.where(mask, z, -jnp.inf)
    m = jnp.max(zm, axis=1, keepdims=True)
    e = jnp.where(mask, jnp.exp(z - m), 0.0)
    lse = jnp.log(jnp.sum(e, axis=1, keepdims=True))
    o_ref[...] = z - m - lse


def _row_spec(hp):
    return pl.BlockSpec((TCB, hp), lambda i: (i, 0))


def _p_spec(hp):
    return pl.BlockSpec((2, TCB, hp), lambda i: (0, i, 0))


def _full_spec(shape):
    return pl.BlockSpec(shape, lambda i: tuple(0 for _ in shape))


def _tc_first(xp, w1p, pd):
    return pl.pallas_call(
        _tc_first_body,
        grid=(GRID,),
        in_specs=[_row_spec(D), _full_spec((D, 32)), _p_spec(16)],
        out_specs=_row_spec(32),
        out_shape=jax.ShapeDtypeStruct((NP, 32), jnp.float32),
    )(xp, w1p, pd)


def _tc_mid(y, p, pd, bp, wp, hin, hout):
    return pl.pallas_call(
        _tc_mid_body,
        grid=(GRID,),
        in_specs=[_row_spec(hin), _p_spec(hin), _p_spec(16),
                  _full_spec((8, hin)), _full_spec((hin, hout))],
        out_specs=_row_spec(hout),
        out_shape=jax.ShapeDtypeStruct((NP, hout), jnp.float32),
    )(y, p, pd, bp, wp)


def _tc_last(y, p, pd, bp):
    return pl.pallas_call(
        _tc_last_body,
        grid=(GRID,),
        in_specs=[_row_spec(16), _p_spec(16), _p_spec(16), _full_spec((8, 16))],
        out_specs=_row_spec(16),
        out_shape=jax.ShapeDtypeStruct((NP, 16), jnp.float32),
    )(y, p, pd, bp)


def _pad2(a, rows, cols):
    return jnp.pad(a, ((0, rows - a.shape[0]), (0, cols - a.shape[1])))


def kernel(x, edge_index, W1, b1, W2, b2, W3, b3):
    src = edge_index[0].astype(jnp.int32)
    dst = edge_index[1].astype(jnp.int32)
    src3 = jnp.concatenate(
        [src, jnp.zeros((EP - E,), jnp.int32)]).reshape(NW, NCH, CH)
    trash = N + jnp.arange(EP - E, dtype=jnp.int32) % (NP - N)
    dst3 = jnp.concatenate([dst, trash]).reshape(NW, NCH, CH)

    xp = _pad2(x, NP, D)
    w1p = _pad2(W1, D, 32)
    w2p = _pad2(W2, 32, 16)
    w3p = _pad2(W3, 16, 16)
    b1p = jnp.tile(jnp.pad(b1, (0, 32 - b1.shape[0]))[None, :], (8, 1))
    b2p = jnp.tile(jnp.pad(b2, (0, 16 - b2.shape[0]))[None, :], (8, 1))
    b3p = jnp.tile(jnp.pad(b3, (0, 16 - b3.shape[0]))[None, :], (8, 1))

    zeros16 = jnp.zeros((NP, 16), jnp.float32)
    zeros32 = jnp.zeros((NP, 32), jnp.float32)
    ones16 = jnp.ones((CH, 16), jnp.float32)

    def dup(y):
        return jnp.broadcast_to(y[None], (2,) + y.shape)

    pd = _make_deg_kernel()(dst3, ones16, zeros16)
    y1 = _tc_first(xp, w1p, pd)
    p1 = _make_layer_kernel(32)(dup(y1), src3, dst3, zeros32)
    y2 = _tc_mid(y1, p1, pd, b1p, w2p, 32, 16)
    p2 = _make_layer_kernel(16)(dup(y2), src3, dst3, zeros16)
    y3 = _tc_mid(y2, p2, pd, b2p, w3p, 16, 16)
    p3 = _make_layer_kernel(16)(dup(y3), src3, dst3, zeros16)
    outp = _tc_last(y3, p3, pd, b3p)
    return outp[:N, :10]


# speedup vs baseline: 1.2411x; 1.2411x over previous
import functools

import jax
import jax.numpy as jnp
from jax import lax
from jax.experimental import pallas as pl
from jax.experimental.pallas import tpu as pltpu
from jax.experimental.pallas import tpu_sc as plsc

N = 10000
NP = 10240
E = 320000
D = 128
CH = 128
NW = 32
NCH = 80
EP = NW * NCH * CH
RPT = NP // 16
TCB = 2048
GRID = NP // TCB


def _sc_mesh():
    return plsc.VectorSubcoreMesh(core_axis_name="c", subcore_axis_name="s")


def _deg_body(dst_hbm, ones_hbm, zeros_hbm, out_hbm, dst_v, ones_v, acc,
              s0, s1, s2, s3):
    sems = (s0, s1, s2, s3)
    cid = lax.axis_index("c")
    sid = lax.axis_index("s")
    wid = sid * 2 + cid
    base = sid * RPT
    pltpu.sync_copy(zeros_hbm.at[pl.ds(base, RPT)], acc.at[pl.ds(base, RPT)])
    pltpu.sync_copy(dst_hbm.at[pl.ds(wid * NCH, NCH)], dst_v)
    pltpu.sync_copy(ones_hbm, ones_v)
    plsc.subcore_barrier()

    def s_fire(j, b):
        pltpu.async_copy(ones_v, acc.at[dst_v.at[j]], sems[b], add=True)

    def s_wait(j, b):
        pltpu.make_async_copy(ones_v, acc.at[dst_v.at[j]], sems[b]).wait()

    for c in range(4):
        s_fire(c, c)

    def body(t, carry):
        for b in range(4):
            j = 4 + 4 * t + b
            s_wait(j - 4, b)
            s_fire(j, b)
        return carry

    lax.fori_loop(0, (NCH - 4) // 4, body, 0)
    for j in range(NCH - 4, NCH):
        s_wait(j, j % 4)

    plsc.subcore_barrier()
    pltpu.sync_copy(acc.at[pl.ds(base, RPT)], out_hbm.at[cid, pl.ds(base, RPT)])


def _make_deg_kernel():
    return pl.kernel(
        _deg_body,
        out_type=jax.ShapeDtypeStruct((2, NP, 16), jnp.float32),
        mesh=_sc_mesh(),
        scratch_types=[
            pltpu.VMEM((NCH, CH), jnp.int32),
            pltpu.VMEM((CH, 16), jnp.float32),
            pltpu.VMEM_SHARED((NP, 16), jnp.float32),
            pltpu.SemaphoreType.DMA,
            pltpu.SemaphoreType.DMA,
            pltpu.SemaphoreType.DMA,
            pltpu.SemaphoreType.DMA,
        ],
        compiler_params=pltpu.CompilerParams(use_tc_tiling_on_sc=False),
    )


DEPTH = 4
NBUF = 2 * DEPTH
C0 = 112
C1 = 48


def _layer_body(y_hbm, src_hbm, dst_hbm, zeros_hbm, out_hbm,
                src_v, dst_v, *rest):
    bufs = rest[:NBUF]
    acc = rest[NBUF]
    gsem = rest[NBUF + 1:2 * NBUF + 1]
    ssem = rest[2 * NBUF + 1:]
    cid = lax.axis_index("c")
    sid = lax.axis_index("s")
    base = sid * RPT
    nch = jnp.where(cid == 0, C0, C1)
    pltpu.sync_copy(zeros_hbm.at[pl.ds(base, RPT)], acc.at[pl.ds(base, RPT)])

    @pl.when(cid == 0)
    def _():
        s0 = sid * C0
        pltpu.sync_copy(src_hbm.at[pl.ds(s0, C0)], src_v)
        pltpu.sync_copy(dst_hbm.at[pl.ds(s0, C0)], dst_v)

    @pl.when(cid == 1)
    def _():
        s1 = 16 * C0 + sid * C1
        pltpu.sync_copy(src_hbm.at[pl.ds(s1, C1)], src_v.at[pl.ds(0, C1)])
        pltpu.sync_copy(dst_hbm.at[pl.ds(s1, C1)], dst_v.at[pl.ds(0, C1)])

    plsc.subcore_barrier()

    def g_fire(j, bi):
        pltpu.async_copy(y_hbm.at[src_v.at[j]], bufs[bi], gsem[bi])

    def g_wait(j, bi):
        pltpu.make_async_copy(y_hbm.at[src_v.at[j]], bufs[bi], gsem[bi]).wait()

    def s_fire(j, bi):
        pltpu.async_copy(bufs[bi], acc.at[dst_v.at[j]], ssem[bi], add=True)

    def s_wait(j, bi):
        pltpu.make_async_copy(bufs[bi], acc.at[dst_v.at[j]], ssem[bi]).wait()

    for c in range(DEPTH):
        g_fire(c, c)
    for j in range(DEPTH):
        g_wait(j, j)
        s_fire(j, j)
        g_fire(j + DEPTH, (j + DEPTH) % NBUF)

    def body(t, carry):
        j0 = DEPTH + t * NBUF
        for b in range(NBUF):
            j = j0 + b
            g_wait(j, (DEPTH + b) % NBUF)
            s_fire(j, (DEPTH + b) % NBUF)
            s_wait(j - DEPTH, b)
            g_fire(j + DEPTH, b)
        return carry

    lax.fori_loop(0, (nch - 2 * DEPTH) // NBUF, body, 0)

    for k in range(DEPTH):
        j = nch - DEPTH + k
        g_wait(j, (DEPTH + k) % NBUF)
        s_fire(j, (DEPTH + k) % NBUF)
        s_wait(j - DEPTH, k)
    for k in range(DEPTH):
        s_wait(nch - DEPTH + k, (DEPTH + k) % NBUF)

    plsc.subcore_barrier()
    pltpu.sync_copy(acc.at[pl.ds(base, RPT)], out_hbm.at[cid, pl.ds(base, RPT)])


def _make_layer_kernel(hp):
    return pl.kernel(
        _layer_body,
        out_type=jax.ShapeDtypeStruct((2, NP, hp), jnp.float32),
        mesh=_sc_mesh(),
        scratch_types=(
            [pltpu.VMEM((C0, CH), jnp.int32),
             pltpu.VMEM((C0, CH), jnp.int32)]
            + [pltpu.VMEM((CH, hp), jnp.float32) for _ in range(NBUF)]
            + [pltpu.VMEM_SHARED((NP, hp), jnp.float32)]
            + [pltpu.SemaphoreType.DMA for _ in range(2 * NBUF)]
        ),
        compiler_params=pltpu.CompilerParams(use_tc_tiling_on_sc=False),
    )


def _dis_of(pd_blk):
    deg = 1.0 + pd_blk[0, :, 0:1] + pd_blk[1, :, 0:1]
    return lax.rsqrt(deg)


def _tc_first_body(x_ref, w_ref, pd_ref, y_ref):
    dis = _dis_of(pd_ref[...])
    y_ref[...] = jnp.dot(x_ref[...], w_ref[...],
                         preferred_element_type=jnp.float32) * dis


def _tc_mid_body(y_ref, p_ref, pd_ref, b_ref, w_ref, o_ref):
    dis = _dis_of(pd_ref[...])
    p = p_ref[...]
    s = y_ref[...] + p[0] + p[1]
    h = jnp.maximum(s * dis + b_ref[0:1, :], 0.0)
    o_ref[...] = jnp.dot(h, w_ref[...],
                         preferred_element_type=jnp.float32) * dis


def _tc_last_body(y_ref, p_ref, pd_ref, b_ref, o_ref):
    dis = _dis_of(pd_ref[...])
    p = p_ref[...]
    z = (y_ref[...] + p[0] + p[1]) * dis + b_ref[0:1, :]
    mask = lax.broadcasted_iota(jnp.int32, z.shape, 1) < 10
    zm = jnp.where(mask, z, -jnp.inf)
    m = jnp.max(zm, axis=1, keepdims=True)
    e = jnp.where(mask, jnp.exp(z - m), 0.0)
    lse = jnp.log(jnp.sum(e, axis=1, keepdims=True))
    o_ref[...] = z - m - lse


def _row_spec(hp):
    return pl.BlockSpec((TCB, hp), lambda i: (i, 0))


def _p_spec(hp):
    return pl.BlockSpec((2, TCB, hp), lambda i: (0, i, 0))


def _full_spec(shape):
    return pl.BlockSpec(shape, lambda i: tuple(0 for _ in shape))


def _tc_first(xp, w1p, pd):
    return pl.pallas_call(
        _tc_first_body,
        grid=(GRID,),
        in_specs=[_row_spec(D), _full_spec((D, 32)), _p_spec(16)],
        out_specs=_row_spec(32),
        out_shape=jax.ShapeDtypeStruct((NP, 32), jnp.float32),
    )(xp, w1p, pd)


def _tc_mid(y, p, pd, bp, wp, hin, hout):
    return pl.pallas_call(
        _tc_mid_body,
        grid=(GRID,),
        in_specs=[_row_spec(hin), _p_spec(hin), _p_spec(16),
                  _full_spec((8, hin)), _full_spec((hin, hout))],
        out_specs=_row_spec(hout),
        out_shape=jax.ShapeDtypeStruct((NP, hout), jnp.float32),
    )(y, p, pd, bp, wp)


def _tc_last(y, p, pd, bp):
    return pl.pallas_call(
        _tc_last_body,
        grid=(GRID,),
        in_specs=[_row_spec(16), _p_spec(16), _p_spec(16), _full_spec((8, 16))],
        out_specs=_row_spec(16),
        out_shape=jax.ShapeDtypeStruct((NP, 16), jnp.float32),
    )(y, p, pd, bp)


def _pad2(a, rows, cols):
    return jnp.pad(a, ((0, rows - a.shape[0]), (0, cols - a.shape[1])))


def kernel(x, edge_index, W1, b1, W2, b2, W3, b3):
    src = edge_index[0].astype(jnp.int32)
    dst = edge_index[1].astype(jnp.int32)
    src3 = jnp.concatenate(
        [src, jnp.zeros((EP - E,), jnp.int32)]).reshape(EP // CH, CH)
    trash = N + jnp.arange(EP - E, dtype=jnp.int32) % (NP - N)
    dst3 = jnp.concatenate([dst, trash]).reshape(EP // CH, CH)

    xp = _pad2(x, NP, D)
    w1p = _pad2(W1, D, 32)
    w2p = _pad2(W2, 32, 16)
    w3p = _pad2(W3, 16, 16)
    b1p = jnp.tile(jnp.pad(b1, (0, 32 - b1.shape[0]))[None, :], (8, 1))
    b2p = jnp.tile(jnp.pad(b2, (0, 16 - b2.shape[0]))[None, :], (8, 1))
    b3p = jnp.tile(jnp.pad(b3, (0, 16 - b3.shape[0]))[None, :], (8, 1))

    zeros16 = jnp.zeros((NP, 16), jnp.float32)
    zeros32 = jnp.zeros((NP, 32), jnp.float32)
    ones16 = jnp.ones((CH, 16), jnp.float32)

    pd = _make_deg_kernel()(dst3, ones16, zeros16)
    y1 = _tc_first(xp, w1p, pd)
    p1 = _make_layer_kernel(32)(y1, src3, dst3, zeros32)
    y2 = _tc_mid(y1, p1, pd, b1p, w2p, 32, 16)
    p2 = _make_layer_kernel(16)(y2, src3, dst3, zeros16)
    y3 = _tc_mid(y2, p2, pd, b2p, w3p, 16, 16)
    p3 = _make_layer_kernel(16)(y3, src3, dst3, zeros16)
    outp = _tc_last(y3, p3, pd, b3p)
    return outp[:N, :10]


# speedup vs baseline: 1.9336x; 1.5580x over previous
import functools

import jax
import jax.numpy as jnp
from jax import lax
from jax.experimental import pallas as pl
from jax.experimental.pallas import tpu as pltpu
from jax.experimental.pallas import tpu_sc as plsc

N = 10000
NP = 10240
E = 320000
D = 128
CH = 128
NW = 32
NCH = 80
EP = NW * NCH * CH
RPT = NP // 16
TCB = 2048
GRID = NP // TCB


def _sc_mesh():
    return plsc.VectorSubcoreMesh(core_axis_name="c", subcore_axis_name="s")


def _deg_body(dst_hbm, ones_hbm, zeros_hbm, out_hbm, dst_v, ones_v, acc,
              s0, s1, s2, s3):
    sems = (s0, s1, s2, s3)
    cid = lax.axis_index("c")
    sid = lax.axis_index("s")
    wid = sid * 2 + cid
    base = sid * RPT
    pltpu.sync_copy(zeros_hbm.at[pl.ds(base, RPT)], acc.at[pl.ds(base, RPT)])
    pltpu.sync_copy(dst_hbm.at[pl.ds(wid * NCH, NCH)], dst_v)
    pltpu.sync_copy(ones_hbm, ones_v)
    plsc.subcore_barrier()

    def s_fire(j, b):
        pltpu.async_copy(ones_v, acc.at[dst_v.at[j]], sems[b], add=True)

    def s_wait(j, b):
        pltpu.make_async_copy(ones_v, acc.at[dst_v.at[j]], sems[b]).wait()

    for c in range(4):
        s_fire(c, c)

    def body(t, carry):
        for b in range(4):
            j = 4 + 4 * t + b
            s_wait(j - 4, b)
            s_fire(j, b)
        return carry

    lax.fori_loop(0, (NCH - 4) // 4, body, 0)
    for j in range(NCH - 4, NCH):
        s_wait(j, j % 4)

    plsc.subcore_barrier()
    pltpu.sync_copy(acc.at[pl.ds(base, RPT)], out_hbm.at[cid, pl.ds(base, RPT)])


def _make_deg_kernel():
    return pl.kernel(
        _deg_body,
        out_type=jax.ShapeDtypeStruct((2, NP, 16), jnp.float32),
        mesh=_sc_mesh(),
        scratch_types=[
            pltpu.VMEM((NCH, CH), jnp.int32),
            pltpu.VMEM((CH, 16), jnp.float32),
            pltpu.VMEM_SHARED((NP, 16), jnp.float32),
            pltpu.SemaphoreType.DMA,
            pltpu.SemaphoreType.DMA,
            pltpu.SemaphoreType.DMA,
            pltpu.SemaphoreType.DMA,
        ],
        compiler_params=pltpu.CompilerParams(use_tc_tiling_on_sc=False),
    )


DEPTH = 4
NBUF = 2 * DEPTH
C0 = 80
C1 = 80


def _layer_body(y_hbm, src_hbm, dst_hbm, zeros_hbm, out_hbm,
                src_v, dst_v, *rest):
    bufs = rest[:NBUF]
    acc = rest[NBUF]
    gsem = rest[NBUF + 1:2 * NBUF + 1]
    ssem = rest[2 * NBUF + 1:]
    cid = lax.axis_index("c")
    sid = lax.axis_index("s")
    base = sid * RPT
    nch = jnp.where(cid == 0, C0, C1)
    pltpu.sync_copy(zeros_hbm.at[pl.ds(base, RPT)], acc.at[pl.ds(base, RPT)])

    @pl.when(cid == 0)
    def _():
        s0 = sid * C0
        pltpu.sync_copy(src_hbm.at[pl.ds(s0, C0)], src_v)
        pltpu.sync_copy(dst_hbm.at[pl.ds(s0, C0)], dst_v)

    @pl.when(cid == 1)
    def _():
        s1 = 16 * C0 + sid * C1
        pltpu.sync_copy(src_hbm.at[pl.ds(s1, C1)], src_v.at[pl.ds(0, C1)])
        pltpu.sync_copy(dst_hbm.at[pl.ds(s1, C1)], dst_v.at[pl.ds(0, C1)])

    plsc.subcore_barrier()

    def g_fire(j, bi):
        pltpu.async_copy(y_hbm.at[src_v.at[j]], bufs[bi], gsem[bi])

    def g_wait(j, bi):
        pltpu.make_async_copy(y_hbm.at[src_v.at[j]], bufs[bi], gsem[bi]).wait()

    def s_fire(j, bi):
        pltpu.async_copy(bufs[bi], acc.at[dst_v.at[j]], ssem[bi], add=True)

    def s_wait(j, bi):
        pltpu.make_async_copy(bufs[bi], acc.at[dst_v.at[j]], ssem[bi]).wait()

    for c in range(DEPTH):
        g_fire(c, c)
    for j in range(DEPTH):
        g_wait(j, j)
        s_fire(j, j)
        g_fire(j + DEPTH, (j + DEPTH) % NBUF)

    def body(t, carry):
        j0 = DEPTH + t * NBUF
        for b in range(NBUF):
            j = j0 + b
            g_wait(j, (DEPTH + b) % NBUF)
            s_fire(j, (DEPTH + b) % NBUF)
            s_wait(j - DEPTH, b)
            g_fire(j + DEPTH, b)
        return carry

    lax.fori_loop(0, (nch - 2 * DEPTH) // NBUF, body, 0)

    for k in range(DEPTH):
        j = nch - DEPTH + k
        g_wait(j, (DEPTH + k) % NBUF)
        s_fire(j, (DEPTH + k) % NBUF)
        s_wait(j - DEPTH, k)
    for k in range(DEPTH):
        s_wait(nch - DEPTH + k, (DEPTH + k) % NBUF)

    plsc.subcore_barrier()
    pltpu.sync_copy(acc.at[pl.ds(base, RPT)], out_hbm.at[cid, pl.ds(base, RPT)])


def _make_layer_kernel(hp):
    return pl.kernel(
        _layer_body,
        out_type=jax.ShapeDtypeStruct((2, NP, hp), jnp.float32),
        mesh=_sc_mesh(),
        scratch_types=(
            [pltpu.VMEM((C0, CH), jnp.int32),
             pltpu.VMEM((C0, CH), jnp.int32)]
            + [pltpu.VMEM((CH, hp), jnp.float32) for _ in range(NBUF)]
            + [pltpu.VMEM_SHARED((NP, hp), jnp.float32)]
            + [pltpu.SemaphoreType.DMA for _ in range(2 * NBUF)]
        ),
        compiler_params=pltpu.CompilerParams(use_tc_tiling_on_sc=False),
    )


def _dis_of(pd_blk):
    deg = 1.0 + pd_blk[0, :, 0:1] + pd_blk[1, :, 0:1]
    return lax.rsqrt(deg)


def _tc_first_body(x_ref, w_ref, pd_ref, y_ref):
    dis = _dis_of(pd_ref[...])
    y_ref[...] = jnp.dot(x_ref[...], w_ref[...],
                         preferred_element_type=jnp.float32) * dis


def _tc_mid_body(y_ref, p_ref, pd_ref, b_ref, w_ref, o_ref):
    dis = _dis_of(pd_ref[...])
    p = p_ref[...]
    s = y_ref[...] + p[0] + p[1]
    h = jnp.maximum(s * dis + b_ref[0:1, :], 0.0)
    o_ref[...] = jnp.dot(h, w_ref[...],
                         preferred_element_type=jnp.float32) * dis


def _tc_last_body(y_ref, p_ref, pd_ref, b_ref, o_ref):
    dis = _dis_of(pd_ref[...])
    p = p_ref[...]
    z = (y_ref[...] + p[0] + p[1]) * dis + b_ref[0:1, :]
    mask = lax.broadcasted_iota(jnp.int32, z.shape, 1) < 10
    zm = jnp.where(mask, z, -jnp.inf)
    m = jnp.max(zm, axis=1, keepdims=True)
    e = jnp.where(mask, jnp.exp(z - m), 0.0)
    lse = jnp.log(jnp.sum(e, axis=1, keepdims=True))
    o_ref[...] = z - m - lse


def _row_spec(hp):
    return pl.BlockSpec((TCB, hp), lambda i: (i, 0))


def _p_spec(hp):
    return pl.BlockSpec((2, TCB, hp), lambda i: (0, i, 0))


def _full_spec(shape):
    return pl.BlockSpec(shape, lambda i: tuple(0 for _ in shape))


def _tc_first(xp, w1p, pd):
    return pl.pallas_call(
        _tc_first_body,
        grid=(GRID,),
        in_specs=[_row_spec(D), _full_spec((D, 32)), _p_spec(16)],
        out_specs=_row_spec(32),
        out_shape=jax.ShapeDtypeStruct((NP, 32), jnp.float32),
    )(xp, w1p, pd)


def _tc_mid(y, p, pd, bp, wp, hin, hout):
    return pl.pallas_call(
        _tc_mid_body,
        grid=(GRID,),
        in_specs=[_row_spec(hin), _p_spec(hin), _p_spec(16),
                  _full_spec((8, hin)), _full_spec((hin, hout))],
        out_specs=_row_spec(hout),
        out_shape=jax.ShapeDtypeStruct((NP, hout), jnp.float32),
    )(y, p, pd, bp, wp)


def _tc_last(y, p, pd, bp):
    return pl.pallas_call(
        _tc_last_body,
        grid=(GRID,),
        in_specs=[_row_spec(16), _p_spec(16), _p_spec(16), _full_spec((8, 16))],
        out_specs=_row_spec(16),
        out_shape=jax.ShapeDtypeStruct((NP, 16), jnp.float32),
    )(y, p, pd, bp)


def _pad2(a, rows, cols):
    return jnp.pad(a, ((0, rows - a.shape[0]), (0, cols - a.shape[1])))


def kernel(x, edge_index, W1, b1, W2, b2, W3, b3):
    src = edge_index[0].astype(jnp.int32)
    dst = edge_index[1].astype(jnp.int32)
    spread = jnp.arange(EP - E, dtype=jnp.int32)
    src3 = jnp.concatenate([src, spread % N]).reshape(EP // CH, CH)
    trash = N + spread % (NP - N)
    dst3 = jnp.concatenate([dst, trash]).reshape(EP // CH, CH)

    xp = _pad2(x, NP, D)
    w1p = _pad2(W1, D, 32)
    w2p = _pad2(W2, 32, 16)
    w3p = _pad2(W3, 16, 16)
    b1p = jnp.tile(jnp.pad(b1, (0, 32 - b1.shape[0]))[None, :], (8, 1))
    b2p = jnp.tile(jnp.pad(b2, (0, 16 - b2.shape[0]))[None, :], (8, 1))
    b3p = jnp.tile(jnp.pad(b3, (0, 16 - b3.shape[0]))[None, :], (8, 1))

    zeros16 = jnp.zeros((NP, 16), jnp.float32)
    zeros32 = jnp.zeros((NP, 32), jnp.float32)
    ones16 = jnp.ones((CH, 16), jnp.float32)

    pd = _make_deg_kernel()(dst3, ones16, zeros16)
    y1 = _tc_first(xp, w1p, pd)
    p1 = _make_layer_kernel(32)(y1, src3, dst3, zeros32)
    y2 = _tc_mid(y1, p1, pd, b1p, w2p, 32, 16)
    p2 = _make_layer_kernel(16)(y2, src3, dst3, zeros16)
    y3 = _tc_mid(y2, p2, pd, b2p, w3p, 16, 16)
    p3 = _make_layer_kernel(16)(y3, src3, dst3, zeros16)
    outp = _tc_last(y3, p3, pd, b3p)
    return outp[:N, :10]


# speedup vs baseline: 2.3122x; 1.1958x over previous
import jax
import jax.numpy as jnp
from jax import lax
from jax.experimental import pallas as pl
from jax.experimental.pallas import tpu as pltpu
from jax.experimental.pallas import tpu_sc as plsc

N = 10000
NP = 10240
E = 320000
D = 128
H = 32
FR = NP * H // 128
CH = 128
RPT = NP // 16
TCB = 512
GRID = FR // TCB


DEPTH = 4
NBUF = 2 * DEPTH
NCH = 80
EP = 32 * NCH * CH


def _sc_mesh():
    return plsc.VectorSubcoreMesh(core_axis_name="c", subcore_axis_name="s")


def _deg_body(dst_hbm, ones_hbm, zeros_hbm, out_hbm, dst_v, ones_v, acc,
              s0, s1, s2, s3):
    sems = (s0, s1, s2, s3)
    cid = lax.axis_index("c")
    sid = lax.axis_index("s")
    wid = sid * 2 + cid
    base = sid * RPT
    pltpu.sync_copy(zeros_hbm.at[pl.ds(base, RPT)], acc.at[pl.ds(base, RPT)])
    pltpu.sync_copy(dst_hbm.at[pl.ds(wid * NCH, NCH)], dst_v)
    pltpu.sync_copy(ones_hbm, ones_v)
    plsc.subcore_barrier()

    def s_fire(j, b):
        pltpu.async_copy(ones_v, acc.at[dst_v.at[j]], sems[b], add=True)

    def s_wait(j, b):
        pltpu.make_async_copy(ones_v, acc.at[dst_v.at[j]], sems[b]).wait()

    for c in range(4):
        s_fire(c, c)

    def body(t, carry):
        for b in range(4):
            j = 4 + 4 * t + b
            s_wait(j - 4, b)
            s_fire(j, b)
        return carry

    lax.fori_loop(0, (NCH - 4) // 4, body, 0)
    for j in range(NCH - 4, NCH):
        s_wait(j, j % 4)

    plsc.subcore_barrier()
    pltpu.sync_copy(acc.at[pl.ds(base, RPT)], out_hbm.at[cid, pl.ds(base, RPT)])


def _make_deg_kernel():
    return pl.kernel(
        _deg_body,
        out_type=jax.ShapeDtypeStruct((2, NP, H), jnp.float32),
        mesh=_sc_mesh(),
        scratch_types=[
            pltpu.VMEM((NCH, CH), jnp.int32),
            pltpu.VMEM((CH, H), jnp.float32),
            pltpu.VMEM_SHARED((NP, H), jnp.float32),
            pltpu.SemaphoreType.DMA,
            pltpu.SemaphoreType.DMA,
            pltpu.SemaphoreType.DMA,
            pltpu.SemaphoreType.DMA,
        ],
        compiler_params=pltpu.CompilerParams(use_tc_tiling_on_sc=False),
    )


def _layer_body(y_hbm, src_hbm, dst_hbm, zeros_hbm, out_hbm,
                src_v, dst_v, *rest):
    bufs = rest[:NBUF]
    acc = rest[NBUF]
    gsem = rest[NBUF + 1:2 * NBUF + 1]
    ssem = rest[2 * NBUF + 1:]
    cid = lax.axis_index("c")
    sid = lax.axis_index("s")
    wid = sid * 2 + cid
    base = sid * RPT
    pltpu.sync_copy(zeros_hbm.at[pl.ds(base, RPT)], acc.at[pl.ds(base, RPT)])
    pltpu.sync_copy(src_hbm.at[pl.ds(wid * NCH, NCH)], src_v)
    pltpu.sync_copy(dst_hbm.at[pl.ds(wid * NCH, NCH)], dst_v)
    plsc.subcore_barrier()

    def g_fire(j, bi):
        pltpu.async_copy(y_hbm.at[src_v.at[j]], bufs[bi], gsem[bi])

    def g_wait(j, bi):
        pltpu.make_async_copy(y_hbm.at[src_v.at[j]], bufs[bi], gsem[bi]).wait()

    def s_fire(j, bi):
        pltpu.async_copy(bufs[bi], acc.at[dst_v.at[j]], ssem[bi], add=True)

    def s_wait(j, bi):
        pltpu.make_async_copy(bufs[bi], acc.at[dst_v.at[j]], ssem[bi]).wait()

    for c in range(DEPTH):
        g_fire(c, c)
    for j in range(DEPTH):
        g_wait(j, j)
        s_fire(j, j)
        g_fire(j + DEPTH, (j + DEPTH) % NBUF)

    def body(t, carry):
        j0 = DEPTH + t * NBUF
        for b in range(NBUF):
            j = j0 + b
            g_wait(j, (DEPTH + b) % NBUF)
            s_fire(j, (DEPTH + b) % NBUF)
            s_wait(j - DEPTH, b)
            g_fire(j + DEPTH, b)
        return carry

    lax.fori_loop(0, (NCH - 2 * DEPTH) // NBUF, body, 0)

    for j in range(NCH - DEPTH, NCH):
        bi = j % NBUF
        g_wait(j, bi)
        s_fire(j, bi)
        s_wait(j - DEPTH, (j - DEPTH) % NBUF)
    for j in range(NCH - DEPTH, NCH):
        s_wait(j, j % NBUF)

    plsc.subcore_barrier()
    pltpu.sync_copy(acc.at[pl.ds(base, RPT)], out_hbm.at[cid, pl.ds(base, RPT)])


def _make_layer_kernel():
    return pl.kernel(
        _layer_body,
        out_type=jax.ShapeDtypeStruct((2, NP, H), jnp.float32),
        mesh=_sc_mesh(),
        scratch_types=(
            [pltpu.VMEM((NCH, CH), jnp.int32),
             pltpu.VMEM((NCH, CH), jnp.int32)]
            + [pltpu.VMEM((CH, H), jnp.float32) for _ in range(NBUF)]
            + [pltpu.VMEM_SHARED((NP, H), jnp.float32)]
            + [pltpu.SemaphoreType.DMA for _ in range(2 * NBUF)]
        ),
        compiler_params=pltpu.CompilerParams(use_tc_tiling_on_sc=False),
    )


def _dis_of(pd_blk):
    return lax.rsqrt(1.0 + pd_blk[0] + pd_blk[1])


def _tc_first_body(x_ref, w_ref, pd_ref, y_ref):
    dis = _dis_of(pd_ref[...])
    y_ref[...] = jnp.dot(x_ref[...], w_ref[...],
                         preferred_element_type=jnp.float32) * dis


def _tc_mid_body(y_ref, p_ref, pd_ref, b_ref, w_ref, o_ref):
    dis = _dis_of(pd_ref[...])
    p = p_ref[...]
    s = y_ref[...] + p[0] + p[1]
    h = jnp.maximum(s * dis + b_ref[0:1, :], 0.0)
    o_ref[...] = jnp.dot(h, w_ref[...],
                         preferred_element_type=jnp.float32) * dis


def _tc_last_body(y_ref, p_ref, pd_ref, b_ref, s_ref, o_ref):
    dis = _dis_of(pd_ref[...])
    p = p_ref[...]
    z = (y_ref[...] + p[0] + p[1]) * dis + b_ref[0:1, :]
    s32 = s_ref[...]
    c = jnp.dot(z, s32, preferred_element_type=jnp.float32) * (1.0 / 32.0)
    zc = z - c
    mask = lax.broadcasted_iota(jnp.int32, z.shape, 1) % 32 < 10
    e = jnp.where(mask, jnp.exp(zc), 0.0)
    ssum = jnp.dot(e, s32, preferred_element_type=jnp.float32)
    o_ref[...] = zc - jnp.log(ssum)


def _fat_spec():
    return pl.BlockSpec((TCB, 128), lambda i: (i, 0))


def _p_spec():
    return pl.BlockSpec((2, TCB, 128), lambda i: (0, i, 0))


def _full_spec(shape):
    return pl.BlockSpec(shape, lambda i: tuple(0 for _ in shape))


def _tc_first(xf, w1bd, pdf):
    return pl.pallas_call(
        _tc_first_body,
        grid=(GRID,),
        in_specs=[pl.BlockSpec((TCB, 512), lambda i: (i, 0)),
                  _full_spec((512, 128)), _p_spec()],
        out_specs=_fat_spec(),
        out_shape=jax.ShapeDtypeStruct((FR, 128), jnp.float32),
    )(xf, w1bd, pdf)


def _tc_mid(yf, pf, pdf, bf, wbd):
    return pl.pallas_call(
        _tc_mid_body,
        grid=(GRID,),
        in_specs=[_fat_spec(), _p_spec(), _p_spec(),
                  _full_spec((8, 128)), _full_spec((128, 128))],
        out_specs=_fat_spec(),
        out_shape=jax.ShapeDtypeStruct((FR, 128), jnp.float32),
    )(yf, pf, pdf, bf, wbd)


def _tc_last(yf, pf, pdf, bf, s32):
    return pl.pallas_call(
        _tc_last_body,
        grid=(GRID,),
        in_specs=[_fat_spec(), _p_spec(), _p_spec(),
                  _full_spec((8, 128)), _full_spec((128, 128))],
        out_specs=_fat_spec(),
        out_shape=jax.ShapeDtypeStruct((FR, 128), jnp.float32),
    )(yf, pf, pdf, bf, s32)


def _bd4(w):
    out = jnp.zeros((4, 32, 4, 32), jnp.float32)
    for i in range(4):
        out = out.at[i, :, i, :].set(w)
    return out.reshape(128, 128)


def _pad2(a, rows, cols):
    return jnp.pad(a, ((0, rows - a.shape[0]), (0, cols - a.shape[1])))


def kernel(x, edge_index, W1, b1, W2, b2, W3, b3):
    src = edge_index[0].astype(jnp.int32)
    dst = edge_index[1].astype(jnp.int32)
    spread = jnp.arange(EP - E, dtype=jnp.int32)
    src3 = jnp.concatenate([src, spread % N]).reshape(EP // CH, CH)
    trash = N + spread % (NP - N)
    dst3 = jnp.concatenate([dst, trash]).reshape(EP // CH, CH)

    xf = _pad2(x, NP, D).reshape(FR, 4 * D)
    w1bd = jnp.zeros((4, D, 4, 32), jnp.float32)
    for i in range(4):
        w1bd = w1bd.at[i, :, i, :].set(_pad2(W1, D, 32))
    w1bd = w1bd.reshape(4 * D, 128)
    w2bd = _bd4(_pad2(W2, 32, 32))
    w3bd = _bd4(_pad2(W3, 32, 32))
    b1f = jnp.tile(jnp.pad(b1, (0, 32 - b1.shape[0])), 4)[None, :].repeat(8, 0)
    b2f = jnp.tile(jnp.pad(b2, (0, 32 - b2.shape[0])), 4)[None, :].repeat(8, 0)
    b3f = jnp.tile(jnp.pad(b3, (0, 32 - b3.shape[0])), 4)[None, :].repeat(8, 0)
    s32 = _bd4(jnp.ones((32, 32), jnp.float32))

    zeros32 = jnp.zeros((NP, H), jnp.float32)
    ones32 = jnp.ones((CH, H), jnp.float32)

    def fat(a):
        return a.reshape(a.shape[:-2] + (FR, 128))

    def unfat(a):
        return a.reshape(NP, H)

    pd = fat(_make_deg_kernel()(dst3, ones32, zeros32))
    layer = _make_layer_kernel()
    y1 = _tc_first(xf, w1bd, pd)
    p1 = fat(layer(unfat(y1), src3, dst3, zeros32))
    y2 = _tc_mid(y1, p1, pd, b1f, w2bd)
    p2 = fat(layer(unfat(y2), src3, dst3, zeros32))
    y3 = _tc_mid(y2, p2, pd, b2f, w3bd)
    p3 = fat(layer(unfat(y3), src3, dst3, zeros32))
    outf = _tc_last(y3, p3, pd, b3f, s32)
    return unfat(outf)[:N, :10]


# speedup vs baseline: 2.3332x; 1.0091x over previous
import jax
import jax.numpy as jnp
from jax import lax
from jax.experimental import pallas as pl
from jax.experimental.pallas import tpu as pltpu
from jax.experimental.pallas import tpu_sc as plsc

N = 10000
NP = 10240
E = 320000
D = 128
H = 32
FR = NP * H // 128
CH = 128
RPT = NP // 16
TCB = 512
GRID = FR // TCB


DEPTH = 4
NBUF = 2 * DEPTH
NCH = 80
EP = 32 * NCH * CH


def _sc_mesh():
    return plsc.VectorSubcoreMesh(core_axis_name="c", subcore_axis_name="s")


def _deg_body(dst_hbm, ones_hbm, zeros_hbm, out_hbm, dst_v, ones_v, acc,
              s0, s1, s2, s3):
    sems = (s0, s1, s2, s3)
    cid = lax.axis_index("c")
    sid = lax.axis_index("s")
    wid = sid * 2 + cid
    base = sid * RPT
    pltpu.sync_copy(zeros_hbm.at[pl.ds(base, RPT)], acc.at[pl.ds(base, RPT)])
    pltpu.sync_copy(dst_hbm.at[pl.ds(wid * NCH, NCH)], dst_v)
    pltpu.sync_copy(ones_hbm, ones_v)
    plsc.subcore_barrier()

    def s_fire(j, b):
        pltpu.async_copy(ones_v, acc.at[dst_v.at[j]], sems[b], add=True)

    def s_wait(j, b):
        pltpu.make_async_copy(ones_v, acc.at[dst_v.at[j]], sems[b]).wait()

    for c in range(4):
        s_fire(c, c)

    def body(t, carry):
        for b in range(4):
            j = 4 + 4 * t + b
            s_wait(j - 4, b)
            s_fire(j, b)
        return carry

    lax.fori_loop(0, (NCH - 4) // 4, body, 0)
    for j in range(NCH - 4, NCH):
        s_wait(j, j % 4)

    plsc.subcore_barrier()
    pltpu.sync_copy(acc.at[pl.ds(base, RPT)], out_hbm.at[cid, pl.ds(base, RPT)])


def _make_deg_kernel():
    return pl.kernel(
        _deg_body,
        out_type=jax.ShapeDtypeStruct((2, NP, H), jnp.float32),
        mesh=_sc_mesh(),
        scratch_types=[
            pltpu.VMEM((NCH, CH), jnp.int32),
            pltpu.VMEM((CH, H), jnp.float32),
            pltpu.VMEM_SHARED((NP, H), jnp.float32),
            pltpu.SemaphoreType.DMA,
            pltpu.SemaphoreType.DMA,
            pltpu.SemaphoreType.DMA,
            pltpu.SemaphoreType.DMA,
        ],
        compiler_params=pltpu.CompilerParams(use_tc_tiling_on_sc=False),
    )


def _layer_body(y_hbm, src_hbm, dst_hbm, zeros_hbm, out_hbm,
                src_v, dst_v, *rest):
    bufs = rest[:NBUF]
    acc = rest[NBUF]
    gsem = rest[NBUF + 1:2 * NBUF + 1]
    ssem = rest[2 * NBUF + 1:]
    cid = lax.axis_index("c")
    sid = lax.axis_index("s")
    wid = sid * 2 + cid
    base = sid * RPT
    pltpu.sync_copy(zeros_hbm.at[pl.ds(base, RPT)], acc.at[pl.ds(base, RPT)])
    pltpu.sync_copy(src_hbm.at[pl.ds(wid * NCH, NCH)], src_v)
    pltpu.sync_copy(dst_hbm.at[pl.ds(wid * NCH, NCH)], dst_v)
    plsc.subcore_barrier()

    def g_fire(j, bi):
        pltpu.async_copy(y_hbm.at[src_v.at[j]], bufs[bi], gsem[bi])

    def g_wait(j, bi):
        pltpu.make_async_copy(y_hbm.at[src_v.at[j]], bufs[bi], gsem[bi]).wait()

    def s_fire(j, bi):
        pltpu.async_copy(bufs[bi], acc.at[dst_v.at[j]], ssem[bi], add=True)

    def s_wait(j, bi):
        pltpu.make_async_copy(bufs[bi], acc.at[dst_v.at[j]], ssem[bi]).wait()

    for c in range(DEPTH):
        g_fire(c, c)
    for j in range(DEPTH):
        g_wait(j, j)
        s_fire(j, j)
        g_fire(j + DEPTH, (j + DEPTH) % NBUF)

    def body(t, carry):
        j0 = DEPTH + t * NBUF
        for b in range(NBUF):
            j = j0 + b
            g_wait(j, (DEPTH + b) % NBUF)
            s_fire(j, (DEPTH + b) % NBUF)
            s_wait(j - DEPTH, b)
            g_fire(j + DEPTH, b)
        return carry

    lax.fori_loop(0, (NCH - 2 * DEPTH) // NBUF, body, 0)

    for j in range(NCH - DEPTH, NCH):
        bi = j % NBUF
        g_wait(j, bi)
        s_fire(j, bi)
        s_wait(j - DEPTH, (j - DEPTH) % NBUF)
    for j in range(NCH - DEPTH, NCH):
        s_wait(j, j % NBUF)

    plsc.subcore_barrier()
    pltpu.sync_copy(acc.at[pl.ds(base, RPT)], out_hbm.at[cid, pl.ds(base, RPT)])


def _make_layer_kernel():
    return pl.kernel(
        _layer_body,
        out_type=jax.ShapeDtypeStruct((2, NP, H), jnp.float32),
        mesh=_sc_mesh(),
        scratch_types=(
            [pltpu.VMEM((NCH, CH), jnp.int32),
             pltpu.VMEM((NCH, CH), jnp.int32)]
            + [pltpu.VMEM((CH, H), jnp.float32) for _ in range(NBUF)]
            + [pltpu.VMEM_SHARED((NP, H), jnp.float32)]
            + [pltpu.SemaphoreType.DMA for _ in range(2 * NBUF)]
        ),
        compiler_params=pltpu.CompilerParams(use_tc_tiling_on_sc=False),
    )


def _dis_of(pd_blk):
    return lax.rsqrt(1.0 + pd_blk[0] + pd_blk[1])


def _tc_matmul_body(x_ref, w_ref, y_ref):
    y_ref[...] = jnp.dot(x_ref[...], w_ref[...],
                         preferred_element_type=jnp.float32)


def _tc_scale_body(z_ref, pd_ref, y_ref):
    y_ref[...] = z_ref[...] * _dis_of(pd_ref[...])


def _tc_mid_body(y_ref, p_ref, pd_ref, b_ref, w_ref, o_ref):
    dis = _dis_of(pd_ref[...])
    p = p_ref[...]
    s = y_ref[...] + p[0] + p[1]
    h = jnp.maximum(s * dis + b_ref[0:1, :], 0.0)
    o_ref[...] = jnp.dot(h, w_ref[...],
                         preferred_element_type=jnp.float32) * dis


def _tc_last_body(y_ref, p_ref, pd_ref, b_ref, s_ref, o_ref):
    dis = _dis_of(pd_ref[...])
    p = p_ref[...]
    z = (y_ref[...] + p[0] + p[1]) * dis + b_ref[0:1, :]
    s32 = s_ref[...]
    c = jnp.dot(z, s32, preferred_element_type=jnp.float32) * (1.0 / 32.0)
    zc = z - c
    mask = lax.broadcasted_iota(jnp.int32, z.shape, 1) % 32 < 10
    e = jnp.where(mask, jnp.exp(zc), 0.0)
    ssum = jnp.dot(e, s32, preferred_element_type=jnp.float32)
    o_ref[...] = zc - jnp.log(ssum)


def _fat_spec():
    return pl.BlockSpec((TCB, 128), lambda i: (i, 0))


def _p_spec():
    return pl.BlockSpec((2, TCB, 128), lambda i: (0, i, 0))


def _full_spec(shape):
    return pl.BlockSpec(shape, lambda i: tuple(0 for _ in shape))


def _tc_first(xf, w1bd, pdf):
    z = pl.pallas_call(
        _tc_matmul_body,
        grid=(GRID,),
        in_specs=[pl.BlockSpec((TCB, 512), lambda i: (i, 0)),
                  _full_spec((512, 128))],
        out_specs=_fat_spec(),
        out_shape=jax.ShapeDtypeStruct((FR, 128), jnp.float32),
    )(xf, w1bd)
    return pl.pallas_call(
        _tc_scale_body,
        grid=(GRID,),
        in_specs=[_fat_spec(), _p_spec()],
        out_specs=_fat_spec(),
        out_shape=jax.ShapeDtypeStruct((FR, 128), jnp.float32),
    )(z, pdf)


def _tc_mid(yf, pf, pdf, bf, wbd):
    return pl.pallas_call(
        _tc_mid_body,
        grid=(GRID,),
        in_specs=[_fat_spec(), _p_spec(), _p_spec(),
                  _full_spec((8, 128)), _full_spec((128, 128))],
        out_specs=_fat_spec(),
        out_shape=jax.ShapeDtypeStruct((FR, 128), jnp.float32),
    )(yf, pf, pdf, bf, wbd)


def _tc_last(yf, pf, pdf, bf, s32):
    return pl.pallas_call(
        _tc_last_body,
        grid=(GRID,),
        in_specs=[_fat_spec(), _p_spec(), _p_spec(),
                  _full_spec((8, 128)), _full_spec((128, 128))],
        out_specs=_fat_spec(),
        out_shape=jax.ShapeDtypeStruct((FR, 128), jnp.float32),
    )(yf, pf, pdf, bf, s32)


def _bd4(w):
    out = jnp.zeros((4, 32, 4, 32), jnp.float32)
    for i in range(4):
        out = out.at[i, :, i, :].set(w)
    return out.reshape(128, 128)


def _pad2(a, rows, cols):
    return jnp.pad(a, ((0, rows - a.shape[0]), (0, cols - a.shape[1])))


def kernel(x, edge_index, W1, b1, W2, b2, W3, b3):
    src = edge_index[0].astype(jnp.int32)
    dst = edge_index[1].astype(jnp.int32)
    spread = jnp.arange(EP - E, dtype=jnp.int32)
    src3 = jnp.concatenate([src, spread % N]).reshape(EP // CH, CH)
    trash = N + spread % (NP - N)
    dst3 = jnp.concatenate([dst, trash]).reshape(EP // CH, CH)

    xf = _pad2(x, NP, D).reshape(FR, 4 * D)
    w1bd = jnp.zeros((4, D, 4, 32), jnp.float32)
    for i in range(4):
        w1bd = w1bd.at[i, :, i, :].set(_pad2(W1, D, 32))
    w1bd = w1bd.reshape(4 * D, 128)
    w2bd = _bd4(_pad2(W2, 32, 32))
    w3bd = _bd4(_pad2(W3, 32, 32))
    b1f = jnp.tile(jnp.pad(b1, (0, 32 - b1.shape[0])), 4)[None, :].repeat(8, 0)
    b2f = jnp.tile(jnp.pad(b2, (0, 32 - b2.shape[0])), 4)[None, :].repeat(8, 0)
    b3f = jnp.tile(jnp.pad(b3, (0, 32 - b3.shape[0])), 4)[None, :].repeat(8, 0)
    s32 = _bd4(jnp.ones((32, 32), jnp.float32))

    zeros32 = jnp.zeros((NP, H), jnp.float32)
    ones32 = jnp.ones((CH, H), jnp.float32)

    def fat(a):
        return a.reshape(a.shape[:-2] + (FR, 128))

    def unfat(a):
        return a.reshape(NP, H)

    pd = fat(_make_deg_kernel()(dst3, ones32, zeros32))
    layer = _make_layer_kernel()
    y1 = _tc_first(xf, w1bd, pd)
    p1 = fat(layer(unfat(y1), src3, dst3, zeros32))
    y2 = _tc_mid(y1, p1, pd, b1f, w2bd)
    p2 = fat(layer(unfat(y2), src3, dst3, zeros32))
    y3 = _tc_mid(y2, p2, pd, b2f, w3bd)
    p3 = fat(layer(unfat(y3), src3, dst3, zeros32))
    outf = _tc_last(y3, p3, pd, b3f, s32)
    return unfat(outf)[:N, :10]
